# baseline (speedup 1.0000x reference)
import functools

import jax
import jax.numpy as jnp
from jax import lax
from jax.experimental import pallas as pl
from jax.experimental.pallas import tpu as pltpu
from jax.experimental.pallas import tpu_sc as plsc

D = 32
N_ROWS = 1000000
N_MAIN = 999936
NCOL = N_MAIN // 128
COLS_W = NCOL // 32
N_EXTRA = NCOL - COLS_W * 32
BLK_WORDS = D * 128
PACK = 128 // D
N_PACKED = N_MAIN // PACK
BPW = 512
NPASS = 2
PASS_B = BPW // NPASS


def _detile_kernel(ut_hbm, it_hbm, uf_hbm, if_hbm,
                   ublk0, ublk1, iblk0, iblk1, uob0, uob1, iob0, iob1,
                   ui_sem, ii_sem, uo_sem, io_sem):
    info = plsc.get_sparse_core_info()
    nc = info.num_cores
    wid = lax.axis_index("s") * nc + lax.axis_index("c")

    lane = lax.iota(jnp.int32, 16)
    bases = [((h * 16) + lane) * D for h in range(8)]
    ublk = (ublk0, ublk1)
    iblk = (iblk0, iblk1)
    uob = (uob0, uob1)
    iob = (iob0, iob1)

    def col_of(k):
        return jnp.where(k < COLS_W, wid + k * 32, COLS_W * 32 + wid)

    def fire_in(k, s):
        coff = pl.multiple_of(col_of(k) * 128, 128)
        pltpu.async_copy(ut_hbm.at[:, pl.ds(coff, 128)], ublk[s], ui_sem)
        pltpu.async_copy(it_hbm.at[:, pl.ds(coff, 128)], iblk[s], ii_sem)

    def fire_out(k, s):
        woff = pl.multiple_of(col_of(k) * BLK_WORDS, BLK_WORDS)
        pltpu.async_copy(uob[s], uf_hbm.at[pl.ds(woff, BLK_WORDS)], uo_sem)
        pltpu.async_copy(iob[s], if_hbm.at[pl.ds(woff, BLK_WORDS)], io_sem)

    def wait_in():
        pltpu.make_async_copy(ut_hbm.at[:, pl.ds(0, 128)], ublk0,
                              ui_sem).wait()
        pltpu.make_async_copy(it_hbm.at[:, pl.ds(0, 128)], iblk0,
                              ii_sem).wait()

    def wait_out():
        pltpu.make_async_copy(uob0, uf_hbm.at[pl.ds(0, BLK_WORDS)],
                              uo_sem).wait()
        pltpu.make_async_copy(iob0, if_hbm.at[pl.ds(0, BLK_WORDS)],
                              io_sem).wait()

    def transpose(s):
        for d in range(D):
            for h in range(8):
                uv = ublk[s][d, pl.ds(h * 16, 16)]
                iv = iblk[s][d, pl.ds(h * 16, 16)]
                idx = bases[h] + d
                plsc.store_scatter(uob[s], [idx], uv)
                plsc.store_scatter(iob[s], [idx], iv)

    def has_col(k):
        return (k < COLS_W) | (wid < N_EXTRA)

    fire_in(0, 0)

    def body(m, carry):
        for s in (0, 1):
            k = m * 2 + s

            wait_in()

            @pl.when(has_col(k + 1))
            def _(k=k, s=s):
                fire_in(k + 1, (s + 1) % 2)

            @pl.when(k >= 2)
            def _():
                wait_out()

            transpose(s)
            fire_out(k, s)
        return carry

    lax.fori_loop(0, COLS_W // 2, body, 0)

    @pl.when(wid < N_EXTRA)
    def _():
        wait_in()
        wait_out()
        transpose(0)
        fire_out(COLS_W, 0)

    wait_out()
    wait_out()


def _gmf_kernel(ux_hbm, ix_hbm, ut_hbm, it_hbm, utail_hbm, itail_hbm, wb_hbm,
                out_hbm,
                uidx_v, iidx_v, ujrow_v, ijrow_v, urows_v, irows_v,
                utail_v, itail_v, wb_v, out_v, usem, isem):
    info = plsc.get_sparse_core_info()
    nc = info.num_cores
    wid = lax.axis_index("s") * nc + lax.axis_index("c")

    pltpu.sync_copy(ux_hbm.at[wid], uidx_v)
    pltpu.sync_copy(ix_hbm.at[wid], iidx_v)
    pltpu.sync_copy(wb_hbm, wb_v)
    pltpu.sync_copy(utail_hbm, utail_v)
    pltpu.sync_copy(itail_hbm, itail_v)

    def rowid_body(t, carry):
        for h in range(8):
            ujrow_v[t, pl.ds(h * 16, 16)] = jnp.minimum(
                uidx_v[t, pl.ds(h * 16, 16)] >> 2, N_PACKED - 1)
            ijrow_v[t, pl.ds(h * 16, 16)] = jnp.minimum(
                iidx_v[t, pl.ds(h * 16, 16)] >> 2, N_PACKED - 1)
        return carry

    lax.fori_loop(0, 4, rowid_body, 0)

    w_lo = wb_v[pl.ds(0, 16)]
    w_hi = wb_v[pl.ds(16, 16)]
    bs = wb_v[pl.ds(D, 16)][0]
    lane = lax.iota(jnp.int32, 16)

    for p in range(NPASS):
        for j in range(PASS_B // 128):
            t = p * (PASS_B // 128) + j
            pltpu.async_copy(ut_hbm.at[ujrow_v.at[t]],
                             urows_v.at[pl.ds(j * 128, 128), :], usem)
            pltpu.async_copy(it_hbm.at[ijrow_v.at[t]],
                             irows_v.at[pl.ds(j * 128, 128), :], isem)
        pltpu.make_async_copy(
            ut_hbm.at[pl.ds(0, PASS_B)], urows_v, usem).wait()
        pltpu.make_async_copy(
            it_hbm.at[pl.ds(0, PASS_B)], irows_v, isem).wait()

        def chunk_body(c, carry, p=p):
            flat = p * PASS_B + c * 16
            t = flat >> 7
            off = flat & 127
            ru = uidx_v[t, pl.ds(off, 16)]
            ri = iidx_v[t, pl.ds(off, 16)]
            uoff = (ru & 3) * D
            ioff = (ri & 3) * D
            rtu = jnp.clip(ru - N_MAIN, 0, 63)
            rti = jnp.clip(ri - N_MAIN, 0, 63)
            utoff = (rtu & 3) * D
            itoff = (rti & 3) * D
            res = jnp.zeros((16,), jnp.float32)
            for k in range(16):
                row = c * 16 + k
                uo = uoff[k]
                io = ioff[k]
                u0 = urows_v[row, pl.ds(uo, 16)]
                u1 = urows_v[row, pl.ds(uo + 16, 16)]
                i0 = irows_v[row, pl.ds(io, 16)]
                i1 = irows_v[row, pl.ds(io + 16, 16)]
                ut0 = utail_v[rtu[k] >> 2, pl.ds(utoff[k], 16)]
                ut1 = utail_v[rtu[k] >> 2, pl.ds(utoff[k] + 16, 16)]
                it0 = itail_v[rti[k] >> 2, pl.ds(itoff[k], 16)]
                it1 = itail_v[rti[k] >> 2, pl.ds(itoff[k] + 16, 16)]
                u_t = ru[k] >= N_MAIN
                i_t = ri[k] >= N_MAIN
                u0 = jnp.where(u_t, ut0, u0)
                u1 = jnp.where(u_t, ut1, u1)
                i0 = jnp.where(i_t, it0, i0)
                i1 = jnp.where(i_t, it1, i1)
                s = jnp.sum(u0 * i0 * w_lo + u1 * i1 * w_hi)
                res = jnp.where(lane == k, s, res)
            out_v[pl.ds(p * PASS_B + c * 16, 16)] = (
                1.0 / (1.0 + jnp.exp(-(res + bs))))
            return carry

        lax.fori_loop(0, PASS_B // 16, chunk_body, 0)

    pltpu.sync_copy(out_v, out_hbm.at[wid])


def kernel(user_x, item_x, user_table, item_table, W, b):
    batch = user_x.shape[0]
    info = plsc.get_sparse_core_info()
    nw = info.num_cores * info.num_subcores
    assert batch == nw * BPW

    mesh = plsc.VectorSubcoreMesh(core_axis_name="c", subcore_axis_name="s")

    detile = functools.partial(
        pl.kernel,
        mesh=mesh,
        compiler_params=pltpu.CompilerParams(needs_layout_passes=False),
        out_type=(jax.ShapeDtypeStruct((N_MAIN * D,), jnp.float32),
                  jax.ShapeDtypeStruct((N_MAIN * D,), jnp.float32)),
        scratch_types=[
            pltpu.VMEM((D, 128), jnp.float32),
            pltpu.VMEM((D, 128), jnp.float32),
            pltpu.VMEM((D, 128), jnp.float32),
            pltpu.VMEM((D, 128), jnp.float32),
            pltpu.VMEM((BLK_WORDS,), jnp.float32),
            pltpu.VMEM((BLK_WORDS,), jnp.float32),
            pltpu.VMEM((BLK_WORDS,), jnp.float32),
            pltpu.VMEM((BLK_WORDS,), jnp.float32),
            pltpu.SemaphoreType.DMA,
            pltpu.SemaphoreType.DMA,
            pltpu.SemaphoreType.DMA,
            pltpu.SemaphoreType.DMA,
        ],
    )(_detile_kernel)
    u_flat, i_flat = detile(user_table.T, item_table.T)

    ux3 = user_x.astype(jnp.int32).reshape(nw, 4, 128)
    ix3 = item_x.astype(jnp.int32).reshape(nw, 4, 128)
    ut_p = u_flat.reshape(N_PACKED, PACK * D)
    it_p = i_flat.reshape(N_PACKED, PACK * D)
    utail = user_table[N_MAIN:].reshape(16, 128)
    itail = item_table[N_MAIN:].reshape(16, 128)
    wb = jnp.concatenate([W.reshape(D), jnp.broadcast_to(b, (D,))])

    run = functools.partial(
        pl.kernel,
        mesh=mesh,
        compiler_params=pltpu.CompilerParams(
            needs_layout_passes=False, use_tc_tiling_on_sc=False),
        out_type=jax.ShapeDtypeStruct((nw, BPW), jnp.float32),
        scratch_types=[
            pltpu.VMEM((4, 128), jnp.int32),
            pltpu.VMEM((4, 128), jnp.int32),
            pltpu.VMEM((4, 128), jnp.int32),
            pltpu.VMEM((4, 128), jnp.int32),
            pltpu.VMEM((PASS_B, PACK * D), jnp.float32),
            pltpu.VMEM((PASS_B, PACK * D), jnp.float32),
            pltpu.VMEM((16, 128), jnp.float32),
            pltpu.VMEM((16, 128), jnp.float32),
            pltpu.VMEM((2 * D,), jnp.float32),
            pltpu.VMEM((BPW,), jnp.float32),
            pltpu.SemaphoreType.DMA,
            pltpu.SemaphoreType.DMA,
        ],
    )(_gmf_kernel)
    out = run(ux3, ix3, ut_p, it_p, utail, itail, wb)
    return out.reshape(batch, 1)


# speedup vs baseline: 1.0036x; 1.0036x over previous
import functools

import jax
import jax.numpy as jnp
from jax import lax
from jax.experimental import pallas as pl
from jax.experimental.pallas import tpu as pltpu
from jax.experimental.pallas import tpu_sc as plsc

D = 32
N_ROWS = 1000000
N_MAIN = 999936
NCOL = N_MAIN // 128
COLS_W = NCOL // 32
N_EXTRA = NCOL - COLS_W * 32
BLK_WORDS = D * 128
PACK = 128 // D
N_PACKED = N_MAIN // PACK
BPW = 512
NPASS = 2
PASS_B = BPW // NPASS


def _detile_kernel(ut_hbm, it_hbm, uf_hbm, if_hbm,
                   ublk0, ublk1, ublk2, ublk3, iblk0, iblk1, iblk2, iblk3,
                   uob0, uob1, uob2, uob3, iob0, iob1, iob2, iob3,
                   ui_sem, ii_sem, uo_sem, io_sem):
    info = plsc.get_sparse_core_info()
    nc = info.num_cores
    wid = lax.axis_index("s") * nc + lax.axis_index("c")

    lane = lax.iota(jnp.int32, 16)
    bases = [((h * 16) + lane) * D for h in range(8)]
    ublk = (ublk0, ublk1, ublk2, ublk3)
    iblk = (iblk0, iblk1, iblk2, iblk3)
    uob = (uob0, uob1, uob2, uob3)
    iob = (iob0, iob1, iob2, iob3)

    def col_of(k):
        return jnp.where(k < COLS_W, wid + k * 32, COLS_W * 32 + wid)

    def fire_in(k, s):
        coff = pl.multiple_of(col_of(k) * 128, 128)
        pltpu.async_copy(ut_hbm.at[:, pl.ds(coff, 128)], ublk[s], ui_sem)
        pltpu.async_copy(it_hbm.at[:, pl.ds(coff, 128)], iblk[s], ii_sem)

    def fire_out(k, s):
        woff = pl.multiple_of(col_of(k) * BLK_WORDS, BLK_WORDS)
        pltpu.async_copy(uob[s], uf_hbm.at[pl.ds(woff, BLK_WORDS)], uo_sem)
        pltpu.async_copy(iob[s], if_hbm.at[pl.ds(woff, BLK_WORDS)], io_sem)

    def wait_in():
        pltpu.make_async_copy(ut_hbm.at[:, pl.ds(0, 128)], ublk0,
                              ui_sem).wait()
        pltpu.make_async_copy(it_hbm.at[:, pl.ds(0, 128)], iblk0,
                              ii_sem).wait()

    def wait_out():
        pltpu.make_async_copy(uob0, uf_hbm.at[pl.ds(0, BLK_WORDS)],
                              uo_sem).wait()
        pltpu.make_async_copy(iob0, if_hbm.at[pl.ds(0, BLK_WORDS)],
                              io_sem).wait()

    def transpose(s):
        for d in range(D):
            for h in range(8):
                uv = ublk[s][d, pl.ds(h * 16, 16)]
                iv = iblk[s][d, pl.ds(h * 16, 16)]
                idx = bases[h] + d
                plsc.store_scatter(uob[s], [idx], uv)
                plsc.store_scatter(iob[s], [idx], iv)

    def can_fire(j):
        return (j < COLS_W) | ((j == COLS_W) & (wid < N_EXTRA))

    for s in range(3):
        fire_in(s, s)

    def body(m, carry):
        for s in (0, 1, 2, 3):
            k = m * 4 + s

            wait_in()

            @pl.when(can_fire(k + 3))
            def _(k=k, s=s):
                fire_in(k + 3, (s + 3) % 4)

            @pl.when(k >= 4)
            def _():
                wait_out()

            transpose(s)
            fire_out(k, s)
        return carry

    lax.fori_loop(0, COLS_W // 4, body, 0)

    @pl.when(wid < N_EXTRA)
    def _():
        wait_in()
        wait_out()
        transpose(0)
        fire_out(COLS_W, 0)

    for _ in range(4):
        wait_out()


def _gmf_kernel(ux_hbm, ix_hbm, ut_hbm, it_hbm, utail_hbm, itail_hbm, wb_hbm,
                out_hbm,
                uidx_v, iidx_v, ujrow_v, ijrow_v, urows_v, irows_v,
                utail_v, itail_v, wb_v, out_v, usem, isem):
    info = plsc.get_sparse_core_info()
    nc = info.num_cores
    wid = lax.axis_index("s") * nc + lax.axis_index("c")

    pltpu.sync_copy(ux_hbm.at[wid], uidx_v)
    pltpu.sync_copy(ix_hbm.at[wid], iidx_v)
    pltpu.sync_copy(wb_hbm, wb_v)
    pltpu.sync_copy(utail_hbm, utail_v)
    pltpu.sync_copy(itail_hbm, itail_v)

    def rowid_body(t, carry):
        for h in range(8):
            ujrow_v[t, pl.ds(h * 16, 16)] = jnp.minimum(
                uidx_v[t, pl.ds(h * 16, 16)] >> 2, N_PACKED - 1)
            ijrow_v[t, pl.ds(h * 16, 16)] = jnp.minimum(
                iidx_v[t, pl.ds(h * 16, 16)] >> 2, N_PACKED - 1)
        return carry

    lax.fori_loop(0, 4, rowid_body, 0)

    w_lo = wb_v[pl.ds(0, 16)]
    w_hi = wb_v[pl.ds(16, 16)]
    bs = wb_v[pl.ds(D, 16)][0]
    lane = lax.iota(jnp.int32, 16)

    for p in range(NPASS):
        for j in range(PASS_B // 128):
            t = p * (PASS_B // 128) + j
            pltpu.async_copy(ut_hbm.at[ujrow_v.at[t]],
                             urows_v.at[pl.ds(j * 128, 128), :], usem)
            pltpu.async_copy(it_hbm.at[ijrow_v.at[t]],
                             irows_v.at[pl.ds(j * 128, 128), :], isem)
        pltpu.make_async_copy(
            ut_hbm.at[pl.ds(0, PASS_B)], urows_v, usem).wait()
        pltpu.make_async_copy(
            it_hbm.at[pl.ds(0, PASS_B)], irows_v, isem).wait()

        def chunk_body(c, carry, p=p):
            flat = p * PASS_B + c * 16
            t = flat >> 7
            off = flat & 127
            ru = uidx_v[t, pl.ds(off, 16)]
            ri = iidx_v[t, pl.ds(off, 16)]
            uoff = (ru & 3) * D
            ioff = (ri & 3) * D
            rtu = jnp.clip(ru - N_MAIN, 0, 63)
            rti = jnp.clip(ri - N_MAIN, 0, 63)
            utoff = (rtu & 3) * D
            itoff = (rti & 3) * D
            res = jnp.zeros((16,), jnp.float32)
            for k in range(16):
                row = c * 16 + k
                uo = uoff[k]
                io = ioff[k]
                u0 = urows_v[row, pl.ds(uo, 16)]
                u1 = urows_v[row, pl.ds(uo + 16, 16)]
                i0 = irows_v[row, pl.ds(io, 16)]
                i1 = irows_v[row, pl.ds(io + 16, 16)]
                ut0 = utail_v[rtu[k] >> 2, pl.ds(utoff[k], 16)]
                ut1 = utail_v[rtu[k] >> 2, pl.ds(utoff[k] + 16, 16)]
                it0 = itail_v[rti[k] >> 2, pl.ds(itoff[k], 16)]
                it1 = itail_v[rti[k] >> 2, pl.ds(itoff[k] + 16, 16)]
                u_t = ru[k] >= N_MAIN
                i_t = ri[k] >= N_MAIN
                u0 = jnp.where(u_t, ut0, u0)
                u1 = jnp.where(u_t, ut1, u1)
                i0 = jnp.where(i_t, it0, i0)
                i1 = jnp.where(i_t, it1, i1)
                s = jnp.sum(u0 * i0 * w_lo + u1 * i1 * w_hi)
                res = jnp.where(lane == k, s, res)
            out_v[pl.ds(p * PASS_B + c * 16, 16)] = (
                1.0 / (1.0 + jnp.exp(-(res + bs))))
            return carry

        lax.fori_loop(0, PASS_B // 16, chunk_body, 0)

    pltpu.sync_copy(out_v, out_hbm.at[wid])


def kernel(user_x, item_x, user_table, item_table, W, b):
    batch = user_x.shape[0]
    info = plsc.get_sparse_core_info()
    nw = info.num_cores * info.num_subcores
    assert batch == nw * BPW

    mesh = plsc.VectorSubcoreMesh(core_axis_name="c", subcore_axis_name="s")

    detile = functools.partial(
        pl.kernel,
        mesh=mesh,
        compiler_params=pltpu.CompilerParams(needs_layout_passes=False),
        out_type=(jax.ShapeDtypeStruct((N_MAIN * D,), jnp.float32),
                  jax.ShapeDtypeStruct((N_MAIN * D,), jnp.float32)),
        scratch_types=(
            [pltpu.VMEM((D, 128), jnp.float32)] * 8
            + [pltpu.VMEM((BLK_WORDS,), jnp.float32)] * 8
            + [pltpu.SemaphoreType.DMA] * 4
        ),
    )(_detile_kernel)
    u_flat, i_flat = detile(user_table.T, item_table.T)

    ux3 = user_x.astype(jnp.int32).reshape(nw, 4, 128)
    ix3 = item_x.astype(jnp.int32).reshape(nw, 4, 128)
    ut_p = u_flat.reshape(N_PACKED, PACK * D)
    it_p = i_flat.reshape(N_PACKED, PACK * D)
    utail = user_table[N_MAIN:].reshape(16, 128)
    itail = item_table[N_MAIN:].reshape(16, 128)
    wb = jnp.concatenate([W.reshape(D), jnp.broadcast_to(b, (D,))])

    run = functools.partial(
        pl.kernel,
        mesh=mesh,
        compiler_params=pltpu.CompilerParams(
            needs_layout_passes=False, use_tc_tiling_on_sc=False),
        out_type=jax.ShapeDtypeStruct((nw, BPW), jnp.float32),
        scratch_types=[
            pltpu.VMEM((4, 128), jnp.int32),
            pltpu.VMEM((4, 128), jnp.int32),
            pltpu.VMEM((4, 128), jnp.int32),
            pltpu.VMEM((4, 128), jnp.int32),
            pltpu.VMEM((PASS_B, PACK * D), jnp.float32),
            pltpu.VMEM((PASS_B, PACK * D), jnp.float32),
            pltpu.VMEM((16, 128), jnp.float32),
            pltpu.VMEM((16, 128), jnp.float32),
            pltpu.VMEM((2 * D,), jnp.float32),
            pltpu.VMEM((BPW,), jnp.float32),
            pltpu.SemaphoreType.DMA,
            pltpu.SemaphoreType.DMA,
        ],
    )(_gmf_kernel)
    out = run(ux3, ix3, ut_p, it_p, utail, itail, wb)
    return out.reshape(batch, 1)


# speedup vs baseline: 1.3833x; 1.3783x over previous
import functools

import jax
import jax.numpy as jnp
from jax import lax
from jax.experimental import pallas as pl
from jax.experimental.pallas import tpu as pltpu
from jax.experimental.pallas import tpu_sc as plsc

D = 32
PACK = 128 // D
N_PACKED = 250000
BPW = 512
NPASS = 2
PASS_B = BPW // NPASS


def _gmf_kernel(ux_hbm, ix_hbm, ut_hbm, it_hbm, wb_hbm, out_hbm,
                uidx_v, iidx_v, ujrow_v, ijrow_v, urows_v, irows_v,
                wb_v, out_v, usem, isem):
    info = plsc.get_sparse_core_info()
    nc = info.num_cores
    wid = lax.axis_index("s") * nc + lax.axis_index("c")

    pltpu.sync_copy(ux_hbm.at[wid], uidx_v)
    pltpu.sync_copy(ix_hbm.at[wid], iidx_v)
    pltpu.sync_copy(wb_hbm, wb_v)

    def rowid_body(t, carry):
        for h in range(8):
            ujrow_v[t, pl.ds(h * 16, 16)] = (
                uidx_v[t, pl.ds(h * 16, 16)] >> 2)
            ijrow_v[t, pl.ds(h * 16, 16)] = (
                iidx_v[t, pl.ds(h * 16, 16)] >> 2)
        return carry

    lax.fori_loop(0, 4, rowid_body, 0)

    w_lo = wb_v[pl.ds(0, 16)]
    w_hi = wb_v[pl.ds(16, 16)]
    bs = wb_v[pl.ds(D, 16)][0]
    lane = lax.iota(jnp.int32, 16)

    for p in range(NPASS):
        for j in range(PASS_B // 128):
            t = p * (PASS_B // 128) + j
            pltpu.async_copy(ut_hbm.at[ujrow_v.at[t]],
                             urows_v.at[pl.ds(j * 128, 128), :], usem)
            pltpu.async_copy(it_hbm.at[ijrow_v.at[t]],
                             irows_v.at[pl.ds(j * 128, 128), :], isem)
        pltpu.make_async_copy(
            ut_hbm.at[pl.ds(0, PASS_B)], urows_v, usem).wait()
        pltpu.make_async_copy(
            it_hbm.at[pl.ds(0, PASS_B)], irows_v, isem).wait()

        def chunk_body(c, carry, p=p):
            flat = p * PASS_B + c * 16
            t = flat >> 7
            off = flat & 127
            ru = uidx_v[t, pl.ds(off, 16)]
            ri = iidx_v[t, pl.ds(off, 16)]
            res = jnp.zeros((16,), jnp.float32)
            uoff = (ru & 3) * D
            ioff = (ri & 3) * D
            for k in range(16):
                row = c * 16 + k
                uo = uoff[k]
                io = ioff[k]
                u0 = urows_v[row, pl.ds(uo, 16)]
                u1 = urows_v[row, pl.ds(uo + 16, 16)]
                i0 = irows_v[row, pl.ds(io, 16)]
                i1 = irows_v[row, pl.ds(io + 16, 16)]
                s = jnp.sum(u0 * i0 * w_lo + u1 * i1 * w_hi)
                res = jnp.where(lane == k, s, res)
            out_v[pl.ds(p * PASS_B + c * 16, 16)] = (
                1.0 / (1.0 + jnp.exp(-(res + bs))))
            return carry

        lax.fori_loop(0, PASS_B // 16, chunk_body, 0)

    pltpu.sync_copy(out_v, out_hbm.at[wid])


def kernel(user_x, item_x, user_table, item_table, W, b):
    batch = user_x.shape[0]
    info = plsc.get_sparse_core_info()
    nw = info.num_cores * info.num_subcores
    assert batch == nw * BPW

    ux3 = user_x.astype(jnp.int32).reshape(nw, 4, 128)
    ix3 = item_x.astype(jnp.int32).reshape(nw, 4, 128)
    ut_p = user_table.reshape(N_PACKED, PACK * D)
    it_p = item_table.reshape(N_PACKED, PACK * D)
    wb = jnp.concatenate([W.reshape(D), jnp.broadcast_to(b, (D,))])

    mesh = plsc.VectorSubcoreMesh(core_axis_name="c", subcore_axis_name="s")
    run = functools.partial(
        pl.kernel,
        mesh=mesh,
        compiler_params=pltpu.CompilerParams(
            needs_layout_passes=False, use_tc_tiling_on_sc=False),
        out_type=jax.ShapeDtypeStruct((nw, BPW), jnp.float32),
        scratch_types=[
            pltpu.VMEM((4, 128), jnp.int32),
            pltpu.VMEM((4, 128), jnp.int32),
            pltpu.VMEM((4, 128), jnp.int32),
            pltpu.VMEM((4, 128), jnp.int32),
            pltpu.VMEM((PASS_B, PACK * D), jnp.float32),
            pltpu.VMEM((PASS_B, PACK * D), jnp.float32),
            pltpu.VMEM((2 * D,), jnp.float32),
            pltpu.VMEM((BPW,), jnp.float32),
            pltpu.SemaphoreType.DMA,
            pltpu.SemaphoreType.DMA,
        ],
    )(_gmf_kernel)
    out = run(ux3, ix3, ut_p, it_p, wb)
    return out.reshape(batch, 1)
